# baseline (speedup 1.0000x reference)
import functools

import jax
import jax.numpy as jnp
from jax import lax
from jax.experimental import pallas as pl
from jax.experimental.pallas import tpu as pltpu
from jax.experimental.pallas import tpu_sc as plsc

C_IN = 100000
D = 64

NC = 2
NS = 16
NW = NC * NS

BT = 128
NT = 200


def _build():
    mesh = plsc.VectorSubcoreMesh(core_axis_name="c", subcore_axis_name="s")

    @functools.partial(
        pl.kernel,
        mesh=mesh,
        compiler_params=pltpu.CompilerParams(
            use_tc_tiling_on_sc=False, needs_layout_passes=False),
        out_type=jax.ShapeDtypeStruct((NT, 8, NW, 8, BT), jnp.float32),
        scratch_types=[
            pltpu.VMEM((NT, BT), jnp.int32),
            pltpu.VMEM((BT, D), jnp.float32),
            pltpu.VMEM((BT, D), jnp.float32),
            pltpu.VMEM((8, 8, BT), jnp.float32),
            pltpu.VMEM((8, 8, BT), jnp.float32),
            pltpu.SemaphoreType.DMA,
            pltpu.SemaphoreType.DMA,
            pltpu.SemaphoreType.DMA,
            pltpu.SemaphoreType.DMA,
        ],
    )
    def emb_kernel(idx_hbm, table_hbm, out_hbm,
                   idx_v, g0, g1, t0, t1, gs0, gs1, os0, os1):
        cid = lax.axis_index("c")
        sid = lax.axis_index("s")
        wid = sid * NC + cid

        gbufs = (g0, g1)
        tbufs = (t0, t1)
        gsems = (gs0, gs1)
        osems = (os0, os1)

        pltpu.sync_copy(idx_hbm.at[wid], idx_v)

        def fire_gather(t, b):
            pltpu.async_copy(table_hbm.at[idx_v.at[t]], gbufs[b], gsems[b])

        def drain_gather(b):
            pltpu.make_async_copy(
                table_hbm.at[pl.ds(0, BT)], gbufs[b], gsems[b]).wait()

        def fire_out(t, b):
            pltpu.async_copy(tbufs[b], out_hbm.at[t, :, wid], osems[b])

        def drain_out(b):
            pltpu.make_async_copy(
                tbufs[b], out_hbm.at[0, :, wid], osems[b]).wait()

        lanes16 = lax.iota(jnp.int32, 16)
        rowvecs = [lanes16 + (l0 * 16) for l0 in range(8)]

        def transpose_block(gb, tb):
            def d_body(d, carry):
                col = jnp.full((16,), d, jnp.int32)
                dst = tb.at[d // 8, d % 8]
                for l0 in range(8):
                    dst[pl.ds(l0 * 16, 16)] = plsc.load_gather(
                        gb, [rowvecs[l0], col])
                return carry
            lax.fori_loop(0, D, d_body, 0)

        fire_gather(0, 0)
        fire_gather(1, 1)

        def chunk_body(c, carry):
            for b in range(2):
                t = 2 * c + b
                drain_gather(b)

                @pl.when(c > 0)
                def _():
                    drain_out(b)

                transpose_block(gbufs[b], tbufs[b])
                fire_out(t, b)
                fire_gather(jnp.minimum(t + 2, NT - 2 + b), b)
            return carry

        lax.fori_loop(0, NT // 2, chunk_body, 0)

        drain_gather(0)
        drain_gather(1)
        drain_out(0)
        drain_out(1)

    return emb_kernel


def kernel(x, W):
    idx = x.T.reshape(NT, NW, BT).transpose(1, 0, 2)
    out5 = _build()(idx, W)
    out = out5.transpose(2, 4, 0, 1, 3).reshape(*x.shape, D)
    return lax.stop_gradient(out)


# speedup vs baseline: 1.3228x; 1.3228x over previous
import functools

import jax
import jax.numpy as jnp
from jax import lax
from jax.experimental import pallas as pl
from jax.experimental.pallas import tpu as pltpu
from jax.experimental.pallas import tpu_sc as plsc

C_IN = 100000
D = 64

NC = 2
NS = 16
NW = NC * NS

BT = 128
NT = 200


def _build():
    mesh = plsc.VectorSubcoreMesh(core_axis_name="c", subcore_axis_name="s")

    @functools.partial(
        pl.kernel,
        mesh=mesh,
        compiler_params=pltpu.CompilerParams(
            use_tc_tiling_on_sc=False, needs_layout_passes=False),
        out_type=jax.ShapeDtypeStruct((NT, 8, NW, 8, BT), jnp.float32),
        scratch_types=[
            pltpu.VMEM((NT, BT), jnp.int32),
            pltpu.VMEM((BT, D), jnp.float32),
            pltpu.VMEM((BT, D), jnp.float32),
            pltpu.VMEM((8, 8, BT), jnp.float32),
            pltpu.VMEM((8, 8, BT), jnp.float32),
            pltpu.SemaphoreType.DMA,
            pltpu.SemaphoreType.DMA,
            pltpu.SemaphoreType.DMA,
            pltpu.SemaphoreType.DMA,
        ],
    )
    def emb_kernel(idx_hbm, table_hbm, out_hbm,
                   idx_v, g0, g1, t0, t1, gs0, gs1, os0, os1):
        cid = lax.axis_index("c")
        sid = lax.axis_index("s")
        wid = sid * NC + cid

        gbufs = (g0, g1)
        tbufs = (t0, t1)
        gsems = (gs0, gs1)
        osems = (os0, os1)

        pltpu.sync_copy(idx_hbm.at[wid], idx_v)

        def fire_gather(t, b):
            pltpu.async_copy(table_hbm.at[idx_v.at[t]], gbufs[b], gsems[b])

        def drain_gather(b):
            pltpu.make_async_copy(
                table_hbm.at[pl.ds(0, BT)], gbufs[b], gsems[b]).wait()

        def fire_out(t, b):
            pltpu.async_copy(tbufs[b], out_hbm.at[t, :, wid], osems[b])

        def drain_out(b):
            pltpu.make_async_copy(
                tbufs[b], out_hbm.at[0, :, wid], osems[b]).wait()

        lanes16 = lax.iota(jnp.int32, 16)
        rowvecs = [lanes16 + (l0 * 16) for l0 in range(8)]

        def transpose_block(gb, tb):
            @plsc.parallel_loop(0, D, unroll=4)
            def _(d):
                col = jnp.full((16,), d, jnp.int32)
                vals = [plsc.load_gather(gb, [rowvecs[l0], col])
                        for l0 in range(8)]
                dst = tb.at[d // 8, d % 8]
                for l0 in range(8):
                    dst[pl.ds(l0 * 16, 16)] = vals[l0]

        fire_gather(0, 0)
        fire_gather(1, 1)

        def chunk_body(c, carry):
            for b in range(2):
                t = 2 * c + b
                drain_gather(b)

                @pl.when(c > 0)
                def _():
                    drain_out(b)

                transpose_block(gbufs[b], tbufs[b])
                fire_out(t, b)
                fire_gather(jnp.minimum(t + 2, NT - 2 + b), b)
            return carry

        lax.fori_loop(0, NT // 2, chunk_body, 0)

        drain_gather(0)
        drain_gather(1)
        drain_out(0)
        drain_out(1)

    return emb_kernel


def kernel(x, W):
    idx = x.T.reshape(NT, NW, BT).transpose(1, 0, 2)
    out5 = _build()(idx, W)
    out = out5.transpose(2, 4, 0, 1, 3).reshape(*x.shape, D)
    return lax.stop_gradient(out)
